# baseline (speedup 1.0000x reference)
import jax
import jax.numpy as jnp
from jax.experimental import pallas as pl
from jax.experimental.pallas import tpu as pltpu


def _xw_kernel(x_ref, w_ref, o_ref):
    o_ref[...] = jnp.dot(
        x_ref[...].astype(jnp.bfloat16), w_ref[...],
        preferred_element_type=jnp.float32).astype(jnp.bfloat16)


def _layer1_kernel(a_ref, z_ref, b1_ref, w2_ref, o_ref):
    h = jnp.dot(a_ref[...], z_ref[...], preferred_element_type=jnp.float32)
    h = h + b1_ref[...]
    h = jnp.where(h > 0, h, 0.2 * h)
    o_ref[...] = jnp.dot(
        h.astype(jnp.bfloat16), w2_ref[...],
        preferred_element_type=jnp.float32).astype(jnp.bfloat16)


def _layer2_kernel(a_ref, u_ref, b2_ref, o_ref):
    y = jnp.dot(a_ref[...], u_ref[...], preferred_element_type=jnp.float32)
    y = y + b2_ref[...]
    m = jnp.max(y, axis=1, keepdims=True)
    e = jnp.exp(y - m)
    o_ref[...] = y - (m + jnp.log(jnp.sum(e, axis=1, keepdims=True)))


def _build_a_hat(edge_index, n):
    src = edge_index[0].astype(jnp.int32)
    dst = edge_index[1].astype(jnp.int32)
    ones = jnp.ones(src.shape, jnp.float32)
    indeg = jnp.zeros((n,), jnp.float32).at[dst].add(ones)
    self_cnt = jnp.zeros((n,), jnp.float32).at[dst].add(
        jnp.where(src == dst, 1.0, 0.0))
    no_self = self_cnt == 0.0
    deg = indeg + jnp.where(no_self, 1.0, 0.0)
    d = jax.lax.rsqrt(deg)
    ar = jnp.arange(n, dtype=jnp.int32)
    rows = jnp.concatenate([dst, ar])
    cols = jnp.concatenate([src, ar])
    vals = jnp.concatenate(
        [d[dst] * d[src], jnp.where(no_self, d * d, 0.0)]).astype(jnp.bfloat16)
    return jnp.zeros((n, n), jnp.bfloat16).at[rows, cols].add(vals)


def kernel(x, edge_index, w1, b1, w2, b2):
    n, fin = x.shape
    hidden = w1.shape[1]
    c = w2.shape[1]
    tm = 512 if n % 512 == 0 else n
    grid = (n // tm,)

    a = _build_a_hat(edge_index, n)
    w1b = w1.astype(jnp.bfloat16)
    w2b = w2.astype(jnp.bfloat16)
    b1f = b1.reshape(1, hidden).astype(jnp.float32)
    b2f = b2.reshape(1, c).astype(jnp.float32)

    z = pl.pallas_call(
        _xw_kernel,
        out_shape=jax.ShapeDtypeStruct((n, hidden), jnp.bfloat16),
        grid=grid,
        in_specs=[pl.BlockSpec((tm, fin), lambda i: (i, 0)),
                  pl.BlockSpec((fin, hidden), lambda i: (0, 0))],
        out_specs=pl.BlockSpec((tm, hidden), lambda i: (i, 0)),
        compiler_params=pltpu.CompilerParams(
            dimension_semantics=("parallel",),
            vmem_limit_bytes=48 << 20,
        ),
        cost_estimate=pl.CostEstimate(
            flops=2 * n * fin * hidden, transcendentals=0,
            bytes_accessed=n * fin * 4 + fin * hidden * 2 + n * hidden * 2),
    )(x, w1b)

    u = pl.pallas_call(
        _layer1_kernel,
        out_shape=jax.ShapeDtypeStruct((n, c), jnp.bfloat16),
        grid=grid,
        in_specs=[pl.BlockSpec((tm, n), lambda i: (i, 0)),
                  pl.BlockSpec((n, hidden), lambda i: (0, 0)),
                  pl.BlockSpec((1, hidden), lambda i: (0, 0)),
                  pl.BlockSpec((hidden, c), lambda i: (0, 0))],
        out_specs=pl.BlockSpec((tm, c), lambda i: (i, 0)),
        compiler_params=pltpu.CompilerParams(
            dimension_semantics=("parallel",),
            vmem_limit_bytes=48 << 20,
        ),
        cost_estimate=pl.CostEstimate(
            flops=2 * n * n * hidden + 2 * n * hidden * c, transcendentals=0,
            bytes_accessed=n * n * 2 + n * hidden * 2 + n * c * 2),
    )(a, z, b1f, w2b)

    out = pl.pallas_call(
        _layer2_kernel,
        out_shape=jax.ShapeDtypeStruct((n, c), jnp.float32),
        grid=grid,
        in_specs=[pl.BlockSpec((tm, n), lambda i: (i, 0)),
                  pl.BlockSpec((n, c), lambda i: (0, 0)),
                  pl.BlockSpec((1, c), lambda i: (0, 0))],
        out_specs=pl.BlockSpec((tm, c), lambda i: (i, 0)),
        compiler_params=pltpu.CompilerParams(
            dimension_semantics=("parallel",),
            vmem_limit_bytes=48 << 20,
        ),
        cost_estimate=pl.CostEstimate(
            flops=2 * n * n * c, transcendentals=n * c + n,
            bytes_accessed=n * n * 2 + n * c * 2 + n * c * 4),
    )(a, u, b2f)

    return out
